# baseline (speedup 1.0000x reference)
import jax
import jax.numpy as jnp
from jax import lax
from jax.experimental import pallas as pl
from jax.experimental.pallas import tpu as pltpu
from jax.experimental.pallas import tpu_sc as plsc

N = 10000
E = 320000
D = 128
SEP = N // 2

NC = 2
NS = 16
NW = NC * NS
B = 112
NB = 90
NB0 = 66
NB1 = 114
EPW = NB * B
EPAD = EPW * NW

ACC_ROWS = 10112
TPW = ACC_ROWS // NS


def _build_t_kernel(emb_ref, wu_ref, bu_ref, wi_ref, bi_ref, out_ref):
    nb = pl.num_programs(0) // 2
    is_user = pl.program_id(0) < nb
    mu = lax.dot_general(wu_ref[...], wi_ref[...], (((0,), (0,)), ((), ())))
    mi = lax.dot_general(wi_ref[...], wu_ref[...], (((0,), (0,)), ((), ())))
    cu = jnp.dot(bu_ref[...], wi_ref[...])
    ci = jnp.dot(bi_ref[...], wu_ref[...])
    m = jnp.where(is_user, mu, mi)
    c = jnp.where(is_user, cu, ci)
    out_ref[...] = jnp.dot(emb_ref[...], m, preferred_element_type=jnp.float32) + c


def _build_t(embeddings, w_user, b_user, w_item, b_item):
    bn = 1000
    nb = N // bn
    return pl.pallas_call(
        _build_t_kernel,
        grid=(2 * nb,),
        in_specs=[
            pl.BlockSpec((bn, D), lambda g: (g % nb, 0)),
            pl.BlockSpec((D, D), lambda g: (0, 0)),
            pl.BlockSpec((1, D), lambda g: (0, 0)),
            pl.BlockSpec((D, D), lambda g: (0, 0)),
            pl.BlockSpec((1, D), lambda g: (0, 0)),
        ],
        out_specs=pl.BlockSpec((bn, D), lambda g: (g, 0)),
        out_shape=jax.ShapeDtypeStruct((2 * N, D), jnp.float32),
    )(embeddings, w_user, b_user.reshape(1, D), w_item, b_item.reshape(1, D))


NBUF = 3


def _sc_kernel(adj_hbm, t_hbm, idx_hbm, zeros_hbm, out_hbm, *scratch):
    idx_v = scratch[0:NBUF]
    w_v = scratch[NBUF:2 * NBUF]
    rows_v = scratch[2 * NBUF:3 * NBUF]
    acc = scratch[3 * NBUF]
    sem_w = scratch[3 * NBUF + 1:4 * NBUF + 1]
    sem_r = scratch[4 * NBUF + 1:5 * NBUF + 1]
    sem_s = scratch[5 * NBUF + 1:6 * NBUF + 1]

    c = lax.axis_index("c")
    s = lax.axis_index("s")

    pltpu.sync_copy(zeros_hbm, acc.at[pl.ds(s * TPW, TPW)])
    plsc.subcore_barrier()

    def fetch(gbase, i, p):
        pltpu.sync_copy(idx_hbm.at[gbase + i], idx_v[p])
        pltpu.async_copy(adj_hbm.at[idx_v[p].at[0]], w_v[p], sem_w[p])
        pltpu.async_copy(t_hbm.at[idx_v[p].at[1]], rows_v[p], sem_r[p])

    def drain_gather(p):
        pltpu.make_async_copy(adj_hbm.at[idx_v[p].at[0]], w_v[p], sem_w[p]).wait()
        pltpu.make_async_copy(t_hbm.at[idx_v[p].at[1]], rows_v[p], sem_r[p]).wait()

    def scale(p):
        def body(g, _):
            wv = w_v[p][pl.ds(g * 16, 16)]
            for j in range(16):
                we = wv[j]
                e = g * 16 + j
                for k in range(D // 16):
                    rows_v[p][e, pl.ds(k * 16, 16)] = (
                        rows_v[p][e, pl.ds(k * 16, 16)] * we)
            return 0
        lax.fori_loop(0, B // 16, body, 0)

    def scatter(p):
        pltpu.async_copy(rows_v[p], acc.at[idx_v[p].at[2]], sem_s[p], add=True)

    def drain_scatter(p):
        pltpu.make_async_copy(rows_v[p], acc.at[idx_v[p].at[2]], sem_s[p]).wait()

    def run_pipeline(nb, gbase):
        for b in range(NBUF - 1):
            fetch(gbase, b, b)

        def bodyn(j, _):
            for b in range(NBUF):
                i = j * NBUF + b
                drain_gather(b)
                scale(b)
                scatter(b)
                inext = i + NBUF - 1
                p2 = (b - 1) % NBUF
                if b == 0:
                    @pl.when((j >= 1) & (inext < nb))
                    def _():
                        drain_scatter(p2)
                else:
                    @pl.when(inext < nb)
                    def _():
                        drain_scatter(p2)

                @pl.when(inext < nb)
                def _():
                    fetch(gbase, inext, p2)
            return 0

        lax.fori_loop(0, nb // NBUF, bodyn, 0)
        for b in range(NBUF):
            drain_scatter(b)

    @pl.when(c == 0)
    def _():
        run_pipeline(NB0, s * NB0)

    @pl.when(c == 1)
    def _():
        run_pipeline(NB1, NS * NB0 + s * NB1)

    plsc.subcore_barrier()

    @pl.when(c == 0)
    def _():
        pltpu.sync_copy(acc.at[pl.ds(s * TPW, TPW)],
                        out_hbm.at[0, pl.ds(s * TPW, TPW)])

    @pl.when(c == 1)
    def _():
        pltpu.sync_copy(acc.at[pl.ds(s * TPW, TPW)],
                        out_hbm.at[1, pl.ds(s * TPW, TPW)])


def _sc_call(adj_matrix, t_table, idx_packed, zeros_stripe):
    mesh = plsc.VectorSubcoreMesh(core_axis_name="c", subcore_axis_name="s",
                                  num_cores=NC, num_subcores=NS)
    run = pl.kernel(
        _sc_kernel,
        out_type=jax.ShapeDtypeStruct((2, ACC_ROWS, D), jnp.float32),
        mesh=mesh,
        scratch_types=(
            [pltpu.VMEM((3, B), jnp.int32)] * NBUF
            + [pltpu.VMEM((B,), jnp.float32)] * NBUF
            + [pltpu.VMEM((B, D), jnp.float32)] * NBUF
            + [pltpu.VMEM_SHARED((ACC_ROWS, D), jnp.float32)]
            + [pltpu.SemaphoreType.DMA] * (3 * NBUF)
        ),
    )
    return run(adj_matrix, t_table, idx_packed, zeros_stripe)


def _sum_kernel(a_ref, b_ref, out_ref):
    out_ref[...] = a_ref[0] + b_ref[0]


def _sum_partials(partials):
    bn = 1000
    return pl.pallas_call(
        _sum_kernel,
        grid=(N // bn,),
        in_specs=[
            pl.BlockSpec((1, bn, D), lambda g: (0, g, 0)),
            pl.BlockSpec((1, bn, D), lambda g: (1, g, 0)),
        ],
        out_specs=pl.BlockSpec((bn, D), lambda g: (g, 0)),
        out_shape=jax.ShapeDtypeStruct((N, D), jnp.float32),
    )(partials, partials)


def kernel(adj_matrix, embeddings, edge_index, W_user, b_user, W_item, b_item):
    u = edge_index[0].astype(jnp.int32)
    v = edge_index[1].astype(jnp.int32)

    fidx = u * N + v
    vroll = jnp.roll(v, -SEP)
    tidx = vroll + jnp.where(jnp.arange(E, dtype=jnp.int32) < E - SEP, 0, N)

    pad = EPAD - E
    fidx = jnp.concatenate([fidx, jnp.zeros((pad,), jnp.int32)])
    tidx = jnp.concatenate([tidx, jnp.zeros((pad,), jnp.int32)])
    uidx = jnp.concatenate([u, jnp.full((pad,), N, jnp.int32)])
    idx_packed = jnp.stack(
        [fidx.reshape(-1, B), tidx.reshape(-1, B), uidx.reshape(-1, B)], axis=1)

    t_table = _build_t(embeddings, W_user, b_user, W_item, b_item)
    zeros_stripe = jnp.zeros((TPW, D), jnp.float32)
    partials = _sc_call(adj_matrix.reshape(-1), t_table, idx_packed, zeros_stripe)
    return _sum_partials(partials)


# speedup vs baseline: 1.0624x; 1.0624x over previous
import jax
import jax.numpy as jnp
from jax import lax
from jax.experimental import pallas as pl
from jax.experimental.pallas import tpu as pltpu
from jax.experimental.pallas import tpu_sc as plsc

N = 10000
E = 320000
D = 128
SEP = N // 2

NC = 2
NS = 16
NW = NC * NS
B = 112
NB = 90
NB0 = 114
NB1 = 66
EPW = NB * B
EPAD = EPW * NW

ACC_ROWS = 10112
TPW = ACC_ROWS // NS


def _build_t_kernel(emb_ref, wu_ref, bu_ref, wi_ref, bi_ref, out_ref):
    nb = pl.num_programs(0) // 2
    is_user = pl.program_id(0) < nb
    mu = lax.dot_general(wu_ref[...], wi_ref[...], (((0,), (0,)), ((), ())))
    mi = lax.dot_general(wi_ref[...], wu_ref[...], (((0,), (0,)), ((), ())))
    cu = jnp.dot(bu_ref[...], wi_ref[...])
    ci = jnp.dot(bi_ref[...], wu_ref[...])
    m = jnp.where(is_user, mu, mi)
    c = jnp.where(is_user, cu, ci)
    out_ref[...] = jnp.dot(emb_ref[...], m, preferred_element_type=jnp.float32) + c


def _build_t(embeddings, w_user, b_user, w_item, b_item):
    bn = 1000
    nb = N // bn
    return pl.pallas_call(
        _build_t_kernel,
        grid=(2 * nb,),
        in_specs=[
            pl.BlockSpec((bn, D), lambda g: (g % nb, 0)),
            pl.BlockSpec((D, D), lambda g: (0, 0)),
            pl.BlockSpec((1, D), lambda g: (0, 0)),
            pl.BlockSpec((D, D), lambda g: (0, 0)),
            pl.BlockSpec((1, D), lambda g: (0, 0)),
        ],
        out_specs=pl.BlockSpec((bn, D), lambda g: (g, 0)),
        out_shape=jax.ShapeDtypeStruct((2 * N, D), jnp.float32),
    )(embeddings, w_user, b_user.reshape(1, D), w_item, b_item.reshape(1, D))


NBUF = 3


def _sc_kernel(adj_hbm, t_hbm, idx_hbm, zeros_hbm, out_hbm, *scratch):
    idx_v = scratch[0:NBUF]
    w_v = scratch[NBUF:2 * NBUF]
    rows_v = scratch[2 * NBUF:3 * NBUF]
    acc = scratch[3 * NBUF]
    sem_w = scratch[3 * NBUF + 1:4 * NBUF + 1]
    sem_r = scratch[4 * NBUF + 1:5 * NBUF + 1]
    sem_s = scratch[5 * NBUF + 1:6 * NBUF + 1]

    c = lax.axis_index("c")
    s = lax.axis_index("s")

    pltpu.sync_copy(zeros_hbm, acc.at[pl.ds(s * TPW, TPW)])
    plsc.subcore_barrier()

    def fetch(gbase, i, p):
        pltpu.sync_copy(idx_hbm.at[gbase + i], idx_v[p])
        pltpu.async_copy(adj_hbm.at[idx_v[p].at[0]], w_v[p], sem_w[p])
        pltpu.async_copy(t_hbm.at[idx_v[p].at[1]], rows_v[p], sem_r[p])

    def drain_gather(p):
        pltpu.make_async_copy(adj_hbm.at[idx_v[p].at[0]], w_v[p], sem_w[p]).wait()
        pltpu.make_async_copy(t_hbm.at[idx_v[p].at[1]], rows_v[p], sem_r[p]).wait()

    def scale(p):
        def body(g, _):
            wv = w_v[p][pl.ds(g * 16, 16)]
            for j in range(16):
                we = wv[j]
                e = g * 16 + j
                for k in range(D // 16):
                    rows_v[p][e, pl.ds(k * 16, 16)] = (
                        rows_v[p][e, pl.ds(k * 16, 16)] * we)
            return 0
        lax.fori_loop(0, B // 16, body, 0)

    def scatter(p):
        pltpu.async_copy(rows_v[p], acc.at[idx_v[p].at[2]], sem_s[p], add=True)

    def drain_scatter(p):
        pltpu.make_async_copy(rows_v[p], acc.at[idx_v[p].at[2]], sem_s[p]).wait()

    def run_pipeline(nb, gbase):
        for b in range(NBUF - 1):
            fetch(gbase, b, b)

        def bodyn(j, _):
            for b in range(NBUF):
                i = j * NBUF + b
                drain_gather(b)
                scale(b)
                scatter(b)
                inext = i + NBUF - 1
                p2 = (b - 1) % NBUF
                if b == 0:
                    @pl.when((j >= 1) & (inext < nb))
                    def _():
                        drain_scatter(p2)
                else:
                    @pl.when(inext < nb)
                    def _():
                        drain_scatter(p2)

                @pl.when(inext < nb)
                def _():
                    fetch(gbase, inext, p2)
            return 0

        lax.fori_loop(0, nb // NBUF, bodyn, 0)
        for b in range(NBUF):
            drain_scatter(b)

    @pl.when(c == 0)
    def _():
        run_pipeline(NB0, s * NB0)

    @pl.when(c == 1)
    def _():
        run_pipeline(NB1, NS * NB0 + s * NB1)

    plsc.subcore_barrier()

    @pl.when(c == 0)
    def _():
        pltpu.sync_copy(acc.at[pl.ds(s * TPW, TPW)],
                        out_hbm.at[0, pl.ds(s * TPW, TPW)])

    @pl.when(c == 1)
    def _():
        pltpu.sync_copy(acc.at[pl.ds(s * TPW, TPW)],
                        out_hbm.at[1, pl.ds(s * TPW, TPW)])


def _sc_call(adj_matrix, t_table, idx_packed, zeros_stripe):
    mesh = plsc.VectorSubcoreMesh(core_axis_name="c", subcore_axis_name="s",
                                  num_cores=NC, num_subcores=NS)
    run = pl.kernel(
        _sc_kernel,
        out_type=jax.ShapeDtypeStruct((2, ACC_ROWS, D), jnp.float32),
        mesh=mesh,
        scratch_types=(
            [pltpu.VMEM((3, B), jnp.int32)] * NBUF
            + [pltpu.VMEM((B,), jnp.float32)] * NBUF
            + [pltpu.VMEM((B, D), jnp.float32)] * NBUF
            + [pltpu.VMEM_SHARED((ACC_ROWS, D), jnp.float32)]
            + [pltpu.SemaphoreType.DMA] * (3 * NBUF)
        ),
    )
    return run(adj_matrix, t_table, idx_packed, zeros_stripe)


def _sum_kernel(a_ref, b_ref, out_ref):
    out_ref[...] = a_ref[0] + b_ref[0]


def _sum_partials(partials):
    bn = 1000
    return pl.pallas_call(
        _sum_kernel,
        grid=(N // bn,),
        in_specs=[
            pl.BlockSpec((1, bn, D), lambda g: (0, g, 0)),
            pl.BlockSpec((1, bn, D), lambda g: (1, g, 0)),
        ],
        out_specs=pl.BlockSpec((bn, D), lambda g: (g, 0)),
        out_shape=jax.ShapeDtypeStruct((N, D), jnp.float32),
    )(partials, partials)


def kernel(adj_matrix, embeddings, edge_index, W_user, b_user, W_item, b_item):
    u = edge_index[0].astype(jnp.int32)
    v = edge_index[1].astype(jnp.int32)

    fidx = u * N + v
    vroll = jnp.roll(v, -SEP)
    tidx = vroll + jnp.where(jnp.arange(E, dtype=jnp.int32) < E - SEP, 0, N)

    pad = EPAD - E
    fidx = jnp.concatenate([fidx, jnp.zeros((pad,), jnp.int32)])
    tidx = jnp.concatenate([tidx, jnp.zeros((pad,), jnp.int32)])
    uidx = jnp.concatenate([u, jnp.full((pad,), N, jnp.int32)])
    idx_packed = jnp.stack(
        [fidx.reshape(-1, B), tidx.reshape(-1, B), uidx.reshape(-1, B)], axis=1)

    t_table = _build_t(embeddings, W_user, b_user, W_item, b_item)
    zeros_stripe = jnp.zeros((TPW, D), jnp.float32)
    partials = _sc_call(adj_matrix.reshape(-1), t_table, idx_packed, zeros_stripe)
    return _sum_partials(partials)
